# baseline (speedup 1.0000x reference)
import functools

import jax
import jax.numpy as jnp
from jax import lax
from jax.experimental import pallas as pl
from jax.experimental.pallas import tpu as pltpu
from jax.experimental.pallas import tpu_sc as plsc

BATCH = 16384
D = 64
NC = 2
NS = 16
L = 16
NW = NC * NS
BPW = BATCH // NW
CHUNK = 128
NCHUNK = BPW // CHUNK

_SCRATCH = lambda: [
    pltpu.VMEM((NCHUNK, CHUNK), jnp.int32),
    pltpu.VMEM((NCHUNK, CHUNK), jnp.int32),
    pltpu.VMEM((BPW, D), jnp.float32),
    pltpu.VMEM((BPW, D), jnp.float32),
    pltpu.VMEM((2 * D + L,), jnp.float32),
    pltpu.VMEM((L * L,), jnp.float32),
    pltpu.VMEM((BPW,), jnp.float32),
    pltpu.SemaphoreType.DMA,
    pltpu.SemaphoreType.DMA,
]


def _ncf_body(user_hbm, item_hbm, ut_hbm, it_hbm, wb_hbm, out_hbm,
                 idx_u, idx_i, rows_u, rows_i, wb_v, part_v, out_v,
                 sem_u, sem_i):
    wid = lax.axis_index("s") * NC + lax.axis_index("c")
    base = wid * BPW

    for c in range(NCHUNK):
        pltpu.sync_copy(user_hbm.at[pl.ds(base + c * CHUNK, CHUNK)], idx_u.at[c])
        pltpu.sync_copy(item_hbm.at[pl.ds(base + c * CHUNK, CHUNK)], idx_i.at[c])
    copies = []
    for c in range(NCHUNK):
        copies.append(pltpu.async_copy(
            ut_hbm.at[idx_u.at[c]], rows_u.at[pl.ds(c * CHUNK, CHUNK)], sem_u))
        copies.append(pltpu.async_copy(
            it_hbm.at[idx_i.at[c]], rows_i.at[pl.ds(c * CHUNK, CHUNK)], sem_i))
    pltpu.sync_copy(wb_hbm, wb_v)

    w_regs = [wb_v[pl.ds(k * L, L)] for k in range(8)]
    lane = lax.iota(jnp.int32, L)
    bias_vec = jnp.where(lane == 0, wb_v[pl.ds(2 * D, L)], 0.0)

    for cp in copies:
        cp.wait()

    def group(g, _):
        i0 = g * L
        for r in range(L):
            i = i0 + r
            acc = bias_vec
            for k in range(4):
                acc = acc + rows_u[i, pl.ds(k * L, L)] * w_regs[k]
            for k in range(4):
                acc = acc + rows_i[i, pl.ds(k * L, L)] * w_regs[4 + k]
            part_v[pl.ds(r * L, L)] = acc
        lane_base = lane * L
        res = plsc.load_gather(part_v, [lane_base])
        for d in range(1, L):
            res = res + plsc.load_gather(part_v, [lane_base + d])
        out_v[pl.ds(i0, L)] = res
        return 0

    lax.fori_loop(0, BPW // L, group, 0)
    pltpu.sync_copy(out_v, out_hbm.at[pl.ds(base, BPW)])


_ncf_predict = pl.kernel(
    _ncf_body,
    out_type=jax.ShapeDtypeStruct((BATCH,), jnp.float32),
    mesh=plsc.VectorSubcoreMesh(core_axis_name="c", subcore_axis_name="s"),
    compiler_params=pltpu.CompilerParams(needs_layout_passes=False,
                                         use_tc_tiling_on_sc=False),
    scratch_types=_SCRATCH(),
)


def kernel(user, item, user_table, item_table, W, b):
    wb = jnp.concatenate([
        W.reshape(-1).astype(jnp.float32),
        b.reshape(-1).astype(jnp.float32),
        jnp.zeros((L - 1,), jnp.float32),
    ])
    return _ncf_predict(user.astype(jnp.int32), item.astype(jnp.int32),
                        user_table, item_table, wb)


# speedup vs baseline: 2.8430x; 2.8430x over previous
import jax
import jax.numpy as jnp
from jax import lax
from jax.experimental import pallas as pl
from jax.experimental.pallas import tpu as pltpu
from jax.experimental.pallas import tpu_sc as plsc

N = 1000000
BATCH = 16384
D = 64
BLK = 2048
GRID = (N + BLK - 1) // BLK

NC = 2
NS = 16
L = 16
NW = NC * NS
BPW = BATCH // NW
CHUNK = 128
NCHUNK = BPW // CHUNK


def _sweep_body(wt_ref, b_ref, ut_ref, it_ref, pu_ref, pi_ref):
    wu = wt_ref[0:D, :]
    wi = wt_ref[D:2 * D, :]
    pu_ref[...] = jnp.sum(ut_ref[...] * wu, axis=0) + b_ref[0]
    pi_ref[...] = jnp.sum(it_ref[...] * wi, axis=0)


_SWEEP = pl.pallas_call(
    _sweep_body,
    grid=(GRID,),
    in_specs=[
        pl.BlockSpec((2 * D, 1), lambda i: (0, 0)),
        pl.BlockSpec(memory_space=pltpu.SMEM),
        pl.BlockSpec((D, BLK), lambda i: (0, i)),
        pl.BlockSpec((D, BLK), lambda i: (0, i)),
    ],
    out_specs=[
        pl.BlockSpec((BLK,), lambda i: (i,)),
        pl.BlockSpec((BLK,), lambda i: (i,)),
    ],
    out_shape=[jax.ShapeDtypeStruct((N,), jnp.float32)] * 2,
)


def _gather_body(user_hbm, item_hbm, pu_hbm, pi_hbm, out_hbm,
                 idx_u, idx_i, val_u, val_i, out_v, sem_u, sem_i):
    wid = lax.axis_index("s") * NC + lax.axis_index("c")
    base = wid * BPW
    for c in range(NCHUNK):
        pltpu.sync_copy(user_hbm.at[pl.ds(base + c * CHUNK, CHUNK)], idx_u.at[c])
        pltpu.sync_copy(item_hbm.at[pl.ds(base + c * CHUNK, CHUNK)], idx_i.at[c])
    copies = []
    for c in range(NCHUNK):
        copies.append(pltpu.async_copy(pu_hbm.at[idx_u.at[c]],
                                       val_u.at[pl.ds(c * CHUNK, CHUNK)], sem_u))
        copies.append(pltpu.async_copy(pi_hbm.at[idx_i.at[c]],
                                       val_i.at[pl.ds(c * CHUNK, CHUNK)], sem_i))
    for cp in copies:
        cp.wait()
    for k in range(BPW // L):
        out_v[pl.ds(k * L, L)] = (val_u[pl.ds(k * L, L)] + val_i[pl.ds(k * L, L)])
    pltpu.sync_copy(out_v, out_hbm.at[pl.ds(base, BPW)])


_GATHER = pl.kernel(
    _gather_body,
    out_type=jax.ShapeDtypeStruct((BATCH,), jnp.float32),
    mesh=plsc.VectorSubcoreMesh(core_axis_name="c", subcore_axis_name="s"),
    compiler_params=pltpu.CompilerParams(needs_layout_passes=False,
                                         use_tc_tiling_on_sc=False),
    scratch_types=[
        pltpu.VMEM((NCHUNK, CHUNK), jnp.int32),
        pltpu.VMEM((NCHUNK, CHUNK), jnp.int32),
        pltpu.VMEM((BPW,), jnp.float32),
        pltpu.VMEM((BPW,), jnp.float32),
        pltpu.VMEM((BPW,), jnp.float32),
        pltpu.SemaphoreType.DMA,
        pltpu.SemaphoreType.DMA,
    ],
)


def kernel(user, item, user_table, item_table, W, b):
    wt = W.reshape(2 * D, 1)
    p_u, p_i = _SWEEP(wt, b, user_table.T, item_table.T)
    return _GATHER(user.astype(jnp.int32), item.astype(jnp.int32), p_u, p_i)


# speedup vs baseline: 4.0635x; 1.4293x over previous
import jax
import jax.numpy as jnp
from jax import lax
from jax.experimental import pallas as pl
from jax.experimental.pallas import tpu as pltpu
from jax.experimental.pallas import tpu_sc as plsc

N = 1000000
BATCH = 16384
D = 64
BLK = 4096
GRID = (N + BLK - 1) // BLK

NC = 2
NS = 16
L = 16
NW = NC * NS
BPW = BATCH // NW
CHUNK = 128
NCHUNK = BPW // CHUNK


def _sweep_body(wt_ref, b_ref, ut_ref, it_ref, pu_ref, pi_ref):
    wu = wt_ref[0:D, :]
    wi = wt_ref[D:2 * D, :]
    pu_ref[...] = jnp.sum(ut_ref[...] * wu, axis=0) + b_ref[0]
    pi_ref[...] = jnp.sum(it_ref[...] * wi, axis=0)


_SWEEP = pl.pallas_call(
    _sweep_body,
    grid=(GRID,),
    in_specs=[
        pl.BlockSpec((2 * D, 1), lambda i: (0, 0)),
        pl.BlockSpec(memory_space=pltpu.SMEM),
        pl.BlockSpec((D, BLK), lambda i: (0, i)),
        pl.BlockSpec((D, BLK), lambda i: (0, i)),
    ],
    out_specs=[
        pl.BlockSpec((BLK,), lambda i: (i,)),
        pl.BlockSpec((BLK,), lambda i: (i,)),
    ],
    out_shape=[jax.ShapeDtypeStruct((N,), jnp.float32)] * 2,
    compiler_params=pltpu.CompilerParams(dimension_semantics=("parallel",)),
)


def _gather_body(user_hbm, item_hbm, pu_hbm, pi_hbm, out_hbm,
                 idx_u, idx_i, val_u, val_i, out_v, sem_u, sem_i):
    wid = lax.axis_index("s") * NC + lax.axis_index("c")
    base = wid * BPW
    for c in range(NCHUNK):
        pltpu.sync_copy(user_hbm.at[pl.ds(base + c * CHUNK, CHUNK)], idx_u.at[c])
        pltpu.sync_copy(item_hbm.at[pl.ds(base + c * CHUNK, CHUNK)], idx_i.at[c])
    copies = []
    for c in range(NCHUNK):
        copies.append(pltpu.async_copy(pu_hbm.at[idx_u.at[c]],
                                       val_u.at[pl.ds(c * CHUNK, CHUNK)], sem_u))
        copies.append(pltpu.async_copy(pi_hbm.at[idx_i.at[c]],
                                       val_i.at[pl.ds(c * CHUNK, CHUNK)], sem_i))
    for cp in copies:
        cp.wait()
    for k in range(BPW // L):
        out_v[pl.ds(k * L, L)] = (val_u[pl.ds(k * L, L)] + val_i[pl.ds(k * L, L)])
    pltpu.sync_copy(out_v, out_hbm.at[pl.ds(base, BPW)])


_GATHER = pl.kernel(
    _gather_body,
    out_type=jax.ShapeDtypeStruct((BATCH,), jnp.float32),
    mesh=plsc.VectorSubcoreMesh(core_axis_name="c", subcore_axis_name="s"),
    compiler_params=pltpu.CompilerParams(needs_layout_passes=False,
                                         use_tc_tiling_on_sc=False),
    scratch_types=[
        pltpu.VMEM((NCHUNK, CHUNK), jnp.int32),
        pltpu.VMEM((NCHUNK, CHUNK), jnp.int32),
        pltpu.VMEM((BPW,), jnp.float32),
        pltpu.VMEM((BPW,), jnp.float32),
        pltpu.VMEM((BPW,), jnp.float32),
        pltpu.SemaphoreType.DMA,
        pltpu.SemaphoreType.DMA,
    ],
)


def kernel(user, item, user_table, item_table, W, b):
    wt = W.reshape(2 * D, 1)
    p_u, p_i = _SWEEP(wt, b, user_table.T, item_table.T)
    return _GATHER(user.astype(jnp.int32), item.astype(jnp.int32), p_u, p_i)


# speedup vs baseline: 5.3382x; 1.3137x over previous
import jax
import jax.numpy as jnp
from jax import lax
from jax.experimental import pallas as pl
from jax.experimental.pallas import tpu as pltpu
from jax.experimental.pallas import tpu_sc as plsc

N = 1000000
BATCH = 16384
D = 64
BLK = 8192
GRID = (N + BLK - 1) // BLK

NC = 2
NS = 16
L = 16
NW = NC * NS
BPW = BATCH // NW
CHUNK = 128
NCHUNK = BPW // CHUNK


def _sweep_body(wt_ref, b_ref, ut_ref, it_ref, pu_ref, pi_ref):
    wu = wt_ref[0:D, :]
    wi = wt_ref[D:2 * D, :]
    pu_ref[...] = jnp.sum(ut_ref[...] * wu, axis=0) + b_ref[0]
    pi_ref[...] = jnp.sum(it_ref[...] * wi, axis=0)


_SWEEP = pl.pallas_call(
    _sweep_body,
    grid=(GRID,),
    in_specs=[
        pl.BlockSpec((2 * D, 1), lambda i: (0, 0)),
        pl.BlockSpec(memory_space=pltpu.SMEM),
        pl.BlockSpec((D, BLK), lambda i: (0, i)),
        pl.BlockSpec((D, BLK), lambda i: (0, i)),
    ],
    out_specs=[
        pl.BlockSpec((BLK,), lambda i: (i,)),
        pl.BlockSpec((BLK,), lambda i: (i,)),
    ],
    out_shape=[jax.ShapeDtypeStruct((N,), jnp.float32)] * 2,
    compiler_params=pltpu.CompilerParams(dimension_semantics=("parallel",)),
)


def _gather_body(user_hbm, item_hbm, pu_hbm, pi_hbm, out_hbm,
                 idx_u, idx_i, val_u, val_i, out_v, sem_u, sem_i):
    wid = lax.axis_index("s") * NC + lax.axis_index("c")
    base = wid * BPW
    for c in range(NCHUNK):
        pltpu.sync_copy(user_hbm.at[pl.ds(base + c * CHUNK, CHUNK)], idx_u.at[c])
        pltpu.sync_copy(item_hbm.at[pl.ds(base + c * CHUNK, CHUNK)], idx_i.at[c])
    copies = []
    for c in range(NCHUNK):
        copies.append(pltpu.async_copy(pu_hbm.at[idx_u.at[c]],
                                       val_u.at[pl.ds(c * CHUNK, CHUNK)], sem_u))
        copies.append(pltpu.async_copy(pi_hbm.at[idx_i.at[c]],
                                       val_i.at[pl.ds(c * CHUNK, CHUNK)], sem_i))
    for cp in copies:
        cp.wait()
    for k in range(BPW // L):
        out_v[pl.ds(k * L, L)] = (val_u[pl.ds(k * L, L)] + val_i[pl.ds(k * L, L)])
    pltpu.sync_copy(out_v, out_hbm.at[pl.ds(base, BPW)])


_GATHER = pl.kernel(
    _gather_body,
    out_type=jax.ShapeDtypeStruct((BATCH,), jnp.float32),
    mesh=plsc.VectorSubcoreMesh(core_axis_name="c", subcore_axis_name="s"),
    compiler_params=pltpu.CompilerParams(needs_layout_passes=False,
                                         use_tc_tiling_on_sc=False),
    scratch_types=[
        pltpu.VMEM((NCHUNK, CHUNK), jnp.int32),
        pltpu.VMEM((NCHUNK, CHUNK), jnp.int32),
        pltpu.VMEM((BPW,), jnp.float32),
        pltpu.VMEM((BPW,), jnp.float32),
        pltpu.VMEM((BPW,), jnp.float32),
        pltpu.SemaphoreType.DMA,
        pltpu.SemaphoreType.DMA,
    ],
)


def kernel(user, item, user_table, item_table, W, b):
    wt = W.reshape(2 * D, 1)
    p_u, p_i = _SWEEP(wt, b, user_table.T, item_table.T)
    return _GATHER(user.astype(jnp.int32), item.astype(jnp.int32), p_u, p_i)


# speedup vs baseline: 6.1849x; 1.1586x over previous
import jax
import jax.numpy as jnp
from jax import lax
from jax.experimental import pallas as pl
from jax.experimental.pallas import tpu as pltpu
from jax.experimental.pallas import tpu_sc as plsc

N = 1000000
BATCH = 16384
D = 64
BLK = 16384
GRID = (N + BLK - 1) // BLK

NC = 2
NS = 16
L = 16
NW = NC * NS
BPW = BATCH // NW
CHUNK = 128
NCHUNK = BPW // CHUNK


def _sweep_body(wt_ref, b_ref, ut_ref, it_ref, pu_ref, pi_ref):
    wu = wt_ref[0:D, :]
    wi = wt_ref[D:2 * D, :]
    pu_ref[...] = jnp.sum(ut_ref[...] * wu, axis=0) + b_ref[0]
    pi_ref[...] = jnp.sum(it_ref[...] * wi, axis=0)


_SWEEP = pl.pallas_call(
    _sweep_body,
    grid=(GRID,),
    in_specs=[
        pl.BlockSpec((2 * D, 1), lambda i: (0, 0)),
        pl.BlockSpec(memory_space=pltpu.SMEM),
        pl.BlockSpec((D, BLK), lambda i: (0, i)),
        pl.BlockSpec((D, BLK), lambda i: (0, i)),
    ],
    out_specs=[
        pl.BlockSpec((BLK,), lambda i: (i,)),
        pl.BlockSpec((BLK,), lambda i: (i,)),
    ],
    out_shape=[jax.ShapeDtypeStruct((N,), jnp.float32)] * 2,
    compiler_params=pltpu.CompilerParams(dimension_semantics=("parallel",)),
)


def _gather_body(user_hbm, item_hbm, pu_hbm, pi_hbm, out_hbm,
                 idx_u, idx_i, val_u, val_i, out_v, sem_u, sem_i):
    wid = lax.axis_index("s") * NC + lax.axis_index("c")
    base = wid * BPW
    for c in range(NCHUNK):
        pltpu.sync_copy(user_hbm.at[pl.ds(base + c * CHUNK, CHUNK)], idx_u.at[c])
        pltpu.sync_copy(item_hbm.at[pl.ds(base + c * CHUNK, CHUNK)], idx_i.at[c])
    copies = []
    for c in range(NCHUNK):
        copies.append(pltpu.async_copy(pu_hbm.at[idx_u.at[c]],
                                       val_u.at[pl.ds(c * CHUNK, CHUNK)], sem_u))
        copies.append(pltpu.async_copy(pi_hbm.at[idx_i.at[c]],
                                       val_i.at[pl.ds(c * CHUNK, CHUNK)], sem_i))
    for cp in copies:
        cp.wait()
    for k in range(BPW // L):
        out_v[pl.ds(k * L, L)] = (val_u[pl.ds(k * L, L)] + val_i[pl.ds(k * L, L)])
    pltpu.sync_copy(out_v, out_hbm.at[pl.ds(base, BPW)])


_GATHER = pl.kernel(
    _gather_body,
    out_type=jax.ShapeDtypeStruct((BATCH,), jnp.float32),
    mesh=plsc.VectorSubcoreMesh(core_axis_name="c", subcore_axis_name="s"),
    compiler_params=pltpu.CompilerParams(needs_layout_passes=False,
                                         use_tc_tiling_on_sc=False),
    scratch_types=[
        pltpu.VMEM((NCHUNK, CHUNK), jnp.int32),
        pltpu.VMEM((NCHUNK, CHUNK), jnp.int32),
        pltpu.VMEM((BPW,), jnp.float32),
        pltpu.VMEM((BPW,), jnp.float32),
        pltpu.VMEM((BPW,), jnp.float32),
        pltpu.SemaphoreType.DMA,
        pltpu.SemaphoreType.DMA,
    ],
)


def kernel(user, item, user_table, item_table, W, b):
    wt = W.reshape(2 * D, 1)
    p_u, p_i = _SWEEP(wt, b, user_table.T, item_table.T)
    return _GATHER(user.astype(jnp.int32), item.astype(jnp.int32), p_u, p_i)
